# baseline (speedup 1.0000x reference)
import functools

import jax
import jax.numpy as jnp
from jax.experimental import pallas as pl


def _body(a_ref, t_ref, o_ref):
    o_ref[...] = jnp.dot(a_ref[...], t_ref[...],
                         preferred_element_type=jnp.float32)


def kernel(alignments, alignment_embeds):
    B, N, K = alignments.shape
    D = alignment_embeds.shape[-1]
    R = B * N
    BLK = 4096
    flat = alignments.reshape(R, K)
    out = pl.pallas_call(
        _body,
        grid=(R // BLK,),
        in_specs=[
            pl.BlockSpec((BLK, K), lambda i: (i, 0)),
            pl.BlockSpec((K, D), lambda i: (0, 0)),
        ],
        out_specs=pl.BlockSpec((BLK, D), lambda i: (i, 0)),
        out_shape=jax.ShapeDtypeStruct((R, D), jnp.float32),
    )(flat, alignment_embeds)
    return out.reshape(B, N, D)
